# Initial kernel scaffold; baseline (speedup 1.0000x reference)
import jax
import jax.numpy as jnp
from jax.experimental import pallas as pl


def kernel(vt_feature, cat_indices, cat_offsets, edge_index, cat_table, fuse_W, user, conv1_Wl, conv1_Wr, conv1_b, conv2_Wl, conv2_Wr, conv2_b):
    raise NotImplementedError("write your pallas kernel here")


# speedup vs baseline: 6.9120x; 6.9120x over previous
import functools

import jax
import jax.numpy as jnp
from jax import lax
from jax.experimental import pallas as pl
from jax.experimental.pallas import tpu as pltpu
from jax.experimental.pallas import tpu_sc as plsc

N_USER = 25000
N_ITEM = 25000
N_NODES = N_USER + N_ITEM
N_EDGES = 800000
VT_DIM = 128
CAT_EMBED_DIM = 32
HIDDEN = 64

NC = 2
NS = 16
LANES = 16

GB = 8
BLOCKS_PT = 49
ROWS_PT = GB * BLOCKS_PT
E_ROWS = NS * ROWS_PT
E_PAD = E_ROWS * 128

HALF = 25000
ACC = 25088
ZPT = ACC // NS
GARB0 = HALF

RB = 384

CAT_ROWS_PW = 8
CAT_PW = CAT_ROWS_PW * 128
CAT_PAD = NC * NS * CAT_PW

_MESH = plsc.VectorSubcoreMesh(core_axis_name="c", subcore_axis_name="s")
_SC_PARAMS = pltpu.CompilerParams(use_tc_tiling_on_sc=False)

_CHUNKS_384 = ((0, 384), (384, 384), (768, 384), (1152, 384), (1536, 32))
_CHUNKS_512 = ((0, 512), (512, 512), (1024, 512), (1536, 32))


def _remap_block(dstb, lidx, cid, sid):
    base = cid * HALF
    garb = GARB0 + sid
    for j in range(GB):
        for m in range(128 // LANES):
            d = dstb[j, pl.ds(m * LANES, LANES)]
            loc = d - base
            ok = (loc >= 0) & (loc < HALF)
            lidx[j, pl.ds(m * LANES, LANES)] = jnp.where(ok, loc, garb)


@functools.partial(
    pl.kernel,
    out_type=jax.ShapeDtypeStruct((NC * ACC, HIDDEN), jnp.float32),
    mesh=_MESH,
    compiler_params=_SC_PARAMS,
    scratch_types=[
        pltpu.VMEM((GB, 128), jnp.int32),
        pltpu.VMEM((GB, 128), jnp.int32),
        pltpu.VMEM((GB, 128), jnp.int32),
        pltpu.VMEM((RB, HIDDEN), jnp.float32),
        pltpu.VMEM_SHARED((ACC, HIDDEN), jnp.float32),
        pltpu.SemaphoreType.DMA,
    ],
)
def _sc_spmv(x_hbm, src_hbm, dst_hbm, z_hbm, out_hbm,
             sidx, dstb, lidx, rows, acc, sem):
    cid = lax.axis_index("c")
    sid = lax.axis_index("s")

    pltpu.sync_copy(z_hbm, rows)
    for off, ln in _CHUNKS_384:
        pltpu.sync_copy(rows.at[pl.ds(0, ln)], acc.at[pl.ds(sid * ZPT + off, ln)])
    plsc.subcore_barrier()

    def gather(j):
        return pltpu.async_copy(x_hbm.at[sidx.at[j]],
                                rows.at[pl.ds((j % 3) * 128, 128)], sem)

    def body(blk, carry):
        row0 = sid * ROWS_PT + blk * GB
        pltpu.sync_copy(src_hbm.at[pl.ds(row0, GB)], sidx)
        pltpu.sync_copy(dst_hbm.at[pl.ds(row0, GB)], dstb)
        _remap_block(dstb, lidx, cid, sid)
        cps = [gather(0), gather(1)]
        for j in range(GB):
            if j + 2 < GB:
                cps.append(gather(j + 2))
            cps[j].wait()
            pltpu.sync_copy(rows.at[pl.ds((j % 3) * 128, 128)],
                            acc.at[lidx.at[j]], add=True)
        return carry

    lax.fori_loop(0, BLOCKS_PT, body, 0)
    plsc.subcore_barrier()

    for off, ln in _CHUNKS_384:
        pltpu.sync_copy(acc.at[pl.ds(sid * ZPT + off, ln)], rows.at[pl.ds(0, ln)])
        pltpu.sync_copy(rows.at[pl.ds(0, ln)],
                        out_hbm.at[pl.ds(cid * ACC + sid * ZPT + off, ln)])


@functools.partial(
    pl.kernel,
    out_type=jax.ShapeDtypeStruct((NC * ACC, LANES), jnp.float32),
    mesh=_MESH,
    compiler_params=_SC_PARAMS,
    scratch_types=[
        pltpu.VMEM((GB, 128), jnp.int32),
        pltpu.VMEM((GB, 128), jnp.int32),
        pltpu.VMEM((128, LANES), jnp.float32),
        pltpu.VMEM((512, LANES), jnp.float32),
        pltpu.VMEM_SHARED((ACC, LANES), jnp.float32),
    ],
)
def _sc_deg(dst_hbm, ones_hbm, z_hbm, out_hbm, dstb, lidx, ones_v, buf, dacc):
    cid = lax.axis_index("c")
    sid = lax.axis_index("s")

    pltpu.sync_copy(ones_hbm, ones_v)
    pltpu.sync_copy(z_hbm, buf)
    for off, ln in _CHUNKS_512:
        pltpu.sync_copy(buf.at[pl.ds(0, ln)], dacc.at[pl.ds(sid * ZPT + off, ln)])
    plsc.subcore_barrier()

    def body(blk, carry):
        row0 = sid * ROWS_PT + blk * GB
        pltpu.sync_copy(dst_hbm.at[pl.ds(row0, GB)], dstb)
        _remap_block(dstb, lidx, cid, sid)
        for j in range(GB):
            pltpu.sync_copy(ones_v, dacc.at[lidx.at[j]], add=True)
        return carry

    lax.fori_loop(0, BLOCKS_PT, body, 0)
    plsc.subcore_barrier()

    for off, ln in _CHUNKS_512:
        pltpu.sync_copy(dacc.at[pl.ds(sid * ZPT + off, ln)], buf.at[pl.ds(0, ln)])
        pltpu.sync_copy(buf.at[pl.ds(0, ln)],
                        out_hbm.at[pl.ds(cid * ACC + sid * ZPT + off, ln)])


@functools.partial(
    pl.kernel,
    out_type=jax.ShapeDtypeStruct((CAT_PAD, CAT_EMBED_DIM), jnp.float32),
    mesh=_MESH,
    compiler_params=_SC_PARAMS,
    scratch_types=[
        pltpu.VMEM((CAT_ROWS_PW, 128), jnp.int32),
        pltpu.VMEM((CAT_PW, CAT_EMBED_DIM), jnp.float32),
        pltpu.SemaphoreType.DMA,
    ],
)
def _sc_cat_gather(table_hbm, idx_hbm, out_hbm, sidx, rows, sem):
    wid = lax.axis_index("s") * NC + lax.axis_index("c")
    pltpu.sync_copy(idx_hbm.at[pl.ds(wid * CAT_ROWS_PW, CAT_ROWS_PW)], sidx)
    cps = [
        pltpu.async_copy(table_hbm.at[sidx.at[j]],
                         rows.at[pl.ds(j * 128, 128)], sem)
        for j in range(CAT_ROWS_PW)
    ]
    for cp in cps:
        cp.wait()
    pltpu.sync_copy(rows, out_hbm.at[pl.ds(wid * CAT_PW, CAT_PW)])


_FB = 1000


def _fuse_body(vt_ref, cat_ref, wv_ref, wc_ref, o_ref):
    dn = (((1,), (1,)), ((), ()))
    o_ref[...] = (
        lax.dot_general(vt_ref[...], wv_ref[...], dn,
                        preferred_element_type=jnp.float32)
        + lax.dot_general(cat_ref[...], wc_ref[...], dn,
                          preferred_element_type=jnp.float32)
    )


def _tc_fuse(vt, cat_emb, wv, wc):
    return pl.pallas_call(
        _fuse_body,
        grid=(N_ITEM // _FB,),
        in_specs=[
            pl.BlockSpec((_FB, VT_DIM), lambda i: (i, 0)),
            pl.BlockSpec((_FB, CAT_EMBED_DIM), lambda i: (i, 0)),
            pl.BlockSpec((HIDDEN, VT_DIM), lambda i: (0, 0)),
            pl.BlockSpec((HIDDEN, CAT_EMBED_DIM), lambda i: (0, 0)),
        ],
        out_specs=pl.BlockSpec((_FB, HIDDEN), lambda i: (i, 0)),
        out_shape=jax.ShapeDtypeStruct((N_ITEM, HIDDEN), jnp.float32),
    )(vt, cat_emb, wv, wc)


def _combine_body(leaky, agg_ref, deg_ref, x_ref, wl_ref, wr_ref, b_ref, o_ref):
    dn = (((1,), (1,)), ((), ()))
    inv = 1.0 / jnp.maximum(deg_ref[:, 0:1], 1.0)
    y = (
        lax.dot_general(agg_ref[...] * inv, wl_ref[...], dn,
                        preferred_element_type=jnp.float32)
        + lax.dot_general(x_ref[...], wr_ref[...], dn,
                          preferred_element_type=jnp.float32)
        + b_ref[...]
    )
    if leaky:
        y = jnp.where(y >= 0.0, y, 0.01 * y)
    o_ref[...] = y


def _tc_combine(agg, deg, x, wl, wr, b, leaky):
    return pl.pallas_call(
        functools.partial(_combine_body, leaky),
        grid=(N_NODES // _FB,),
        in_specs=[
            pl.BlockSpec((_FB, HIDDEN), lambda i: (i, 0)),
            pl.BlockSpec((_FB, LANES), lambda i: (i, 0)),
            pl.BlockSpec((_FB, HIDDEN), lambda i: (i, 0)),
            pl.BlockSpec((HIDDEN, HIDDEN), lambda i: (0, 0)),
            pl.BlockSpec((HIDDEN, HIDDEN), lambda i: (0, 0)),
            pl.BlockSpec((1, HIDDEN), lambda i: (0, 0)),
        ],
        out_specs=pl.BlockSpec((_FB, HIDDEN), lambda i: (i, 0)),
        out_shape=jax.ShapeDtypeStruct((N_NODES, HIDDEN), jnp.float32),
    )(agg, deg, x, wl, wr, b)


def _unseam(a):
    return jnp.concatenate([a[:HALF], a[ACC:ACC + HALF]], axis=0)


def kernel(vt_feature, cat_indices, cat_offsets, edge_index, cat_table,
           fuse_W, user, conv1_Wl, conv1_Wr, conv1_b, conv2_Wl, conv2_Wr,
           conv2_b):
    del cat_offsets

    src = edge_index[0].astype(jnp.int32)
    dst = edge_index[1].astype(jnp.int32)
    src2d = jnp.pad(src, (0, E_PAD - N_EDGES)).reshape(E_ROWS, 128)
    dst2d = jnp.pad(dst, (0, E_PAD - N_EDGES),
                    constant_values=2 ** 29).reshape(E_ROWS, 128)
    cat2d = jnp.pad(cat_indices.astype(jnp.int32),
                    (0, CAT_PAD - N_ITEM)).reshape(CAT_PAD // 128, 128)

    z64 = jnp.zeros((RB, HIDDEN), jnp.float32)
    z16 = jnp.zeros((512, LANES), jnp.float32)
    ones16 = jnp.ones((128, LANES), jnp.float32)

    cat_emb = _sc_cat_gather(cat_table, cat2d)
    deg = _unseam(_sc_deg(dst2d, ones16, z16))

    wv = fuse_W[:, :VT_DIM]
    wc = fuse_W[:, VT_DIM:]
    item_feat = _tc_fuse(vt_feature, cat_emb, wv, wc)
    x0 = jnp.concatenate([user, item_feat], axis=0)

    b1 = conv1_b.reshape(1, HIDDEN)
    b2 = conv2_b.reshape(1, HIDDEN)

    agg1 = _unseam(_sc_spmv(x0, src2d, dst2d, z64))
    x1 = _tc_combine(agg1, deg, x0, conv1_Wl, conv1_Wr, b1, leaky=True)
    agg2 = _unseam(_sc_spmv(x1, src2d, dst2d, z64))
    x2 = _tc_combine(agg2, deg, x1, conv2_Wl, conv2_Wr, b2, leaky=False)
    return x2


# speedup vs baseline: 7.9962x; 1.1569x over previous
import functools

import jax
import jax.numpy as jnp
from jax import lax
from jax.experimental import pallas as pl
from jax.experimental.pallas import tpu as pltpu
from jax.experimental.pallas import tpu_sc as plsc

N_USER = 25000
N_ITEM = 25000
N_NODES = N_USER + N_ITEM
N_EDGES = 800000
VT_DIM = 128
CAT_EMBED_DIM = 32
HIDDEN = 64

NC = 2
NS = 16
LANES = 16

GB = 8
BLOCKS_PT = 49
ROWS_PT = GB * BLOCKS_PT
E_ROWS = NS * ROWS_PT
E_PAD = E_ROWS * 128

HALF = 25000
ACC = 25088
ZPT = ACC // NS
GARB0 = HALF

FH = HIDDEN // NC
ACC2 = 50176
ZPT2 = ACC2 // NS
GARB2 = N_NODES

RB = 384
RB2 = 512

CAT_ROWS_PW = 8
CAT_PW = CAT_ROWS_PW * 128
CAT_PAD = NC * NS * CAT_PW

_MESH = plsc.VectorSubcoreMesh(core_axis_name="c", subcore_axis_name="s")
_SC_PARAMS = pltpu.CompilerParams(use_tc_tiling_on_sc=False)

_CHUNKS_512 = ((0, 512), (512, 512), (1024, 512), (1536, 32))
_CHUNKS_3136 = tuple((i * 384, 384) for i in range(8)) + ((3072, 64),)


def _remap_block(dstb, lidx, cid, sid):
    base = cid * HALF
    garb = GARB0 + sid
    for j in range(GB):
        for m in range(128 // LANES):
            d = dstb[j, pl.ds(m * LANES, LANES)]
            loc = d - base
            ok = (loc >= 0) & (loc < HALF)
            lidx[j, pl.ds(m * LANES, LANES)] = jnp.where(ok, loc, garb)


def _remap_full(sidxb, dstb, gidx, lidx, cid, sid):
    garb = GARB2 + sid
    base = cid * N_NODES
    for j in range(GB):
        for m in range(128 // LANES):
            sl = pl.ds(m * LANES, LANES)
            d = dstb[j, sl]
            lidx[j, sl] = jnp.where(d < N_NODES, d, garb)
            gidx[j, sl] = sidxb[j, sl] + base


@functools.partial(
    pl.kernel,
    out_type=jax.ShapeDtypeStruct((NC * ACC2, FH), jnp.float32),
    mesh=_MESH,
    compiler_params=_SC_PARAMS,
    scratch_types=[
        pltpu.VMEM((GB, 128), jnp.int32),
        pltpu.VMEM((GB, 128), jnp.int32),
        pltpu.VMEM((GB, 128), jnp.int32),
        pltpu.VMEM((GB, 128), jnp.int32),
        pltpu.VMEM((RB2, FH), jnp.float32),
        pltpu.VMEM_SHARED((ACC2, FH), jnp.float32),
        pltpu.SemaphoreType.DMA,
    ],
)
def _sc_spmv(xs_hbm, src_hbm, dst_hbm, z_hbm, out_hbm,
             sidx, dstb, gidx, lidx, rows, acc, sem):
    cid = lax.axis_index("c")
    sid = lax.axis_index("s")

    pltpu.sync_copy(z_hbm, rows.at[pl.ds(0, 384)])
    for off, ln in _CHUNKS_3136:
        pltpu.sync_copy(rows.at[pl.ds(0, ln)],
                        acc.at[pl.ds(sid * ZPT2 + off, ln)])
    plsc.subcore_barrier()

    def gather(j):
        return pltpu.async_copy(xs_hbm.at[gidx.at[j]],
                                rows.at[pl.ds((j % 4) * 128, 128)], sem)

    def body(blk, carry):
        row0 = sid * ROWS_PT + blk * GB
        pltpu.sync_copy(src_hbm.at[pl.ds(row0, GB)], sidx)
        pltpu.sync_copy(dst_hbm.at[pl.ds(row0, GB)], dstb)
        _remap_full(sidx, dstb, gidx, lidx, cid, sid)
        cps = [gather(0), gather(1), gather(2)]
        for j in range(GB):
            if j + 3 < GB:
                cps.append(gather(j + 3))
            cps[j].wait()
            pltpu.sync_copy(rows.at[pl.ds((j % 4) * 128, 128)],
                            acc.at[lidx.at[j]], add=True)
        return carry

    lax.fori_loop(0, BLOCKS_PT, body, 0)
    plsc.subcore_barrier()

    for off, ln in _CHUNKS_3136:
        pltpu.sync_copy(acc.at[pl.ds(sid * ZPT2 + off, ln)],
                        rows.at[pl.ds(0, ln)])
        pltpu.sync_copy(rows.at[pl.ds(0, ln)],
                        out_hbm.at[pl.ds(cid * ACC2 + sid * ZPT2 + off, ln)])


@functools.partial(
    pl.kernel,
    out_type=jax.ShapeDtypeStruct((NC * ACC, LANES), jnp.float32),
    mesh=_MESH,
    compiler_params=_SC_PARAMS,
    scratch_types=[
        pltpu.VMEM((GB, 128), jnp.int32),
        pltpu.VMEM((GB, 128), jnp.int32),
        pltpu.VMEM((128, LANES), jnp.float32),
        pltpu.VMEM((512, LANES), jnp.float32),
        pltpu.VMEM_SHARED((ACC, LANES), jnp.float32),
    ],
)
def _sc_deg(dst_hbm, ones_hbm, z_hbm, out_hbm, dstb, lidx, ones_v, buf, dacc):
    cid = lax.axis_index("c")
    sid = lax.axis_index("s")

    pltpu.sync_copy(ones_hbm, ones_v)
    pltpu.sync_copy(z_hbm, buf)
    for off, ln in _CHUNKS_512:
        pltpu.sync_copy(buf.at[pl.ds(0, ln)], dacc.at[pl.ds(sid * ZPT + off, ln)])
    plsc.subcore_barrier()

    def body(blk, carry):
        row0 = sid * ROWS_PT + blk * GB
        pltpu.sync_copy(dst_hbm.at[pl.ds(row0, GB)], dstb)
        _remap_block(dstb, lidx, cid, sid)
        for j in range(GB):
            pltpu.sync_copy(ones_v, dacc.at[lidx.at[j]], add=True)
        return carry

    lax.fori_loop(0, BLOCKS_PT, body, 0)
    plsc.subcore_barrier()

    for off, ln in _CHUNKS_512:
        pltpu.sync_copy(dacc.at[pl.ds(sid * ZPT + off, ln)], buf.at[pl.ds(0, ln)])
        pltpu.sync_copy(buf.at[pl.ds(0, ln)],
                        out_hbm.at[pl.ds(cid * ACC + sid * ZPT + off, ln)])


@functools.partial(
    pl.kernel,
    out_type=jax.ShapeDtypeStruct((CAT_PAD, CAT_EMBED_DIM), jnp.float32),
    mesh=_MESH,
    compiler_params=_SC_PARAMS,
    scratch_types=[
        pltpu.VMEM((CAT_ROWS_PW, 128), jnp.int32),
        pltpu.VMEM((CAT_PW, CAT_EMBED_DIM), jnp.float32),
        pltpu.SemaphoreType.DMA,
    ],
)
def _sc_cat_gather(table_hbm, idx_hbm, out_hbm, sidx, rows, sem):
    wid = lax.axis_index("s") * NC + lax.axis_index("c")
    pltpu.sync_copy(idx_hbm.at[pl.ds(wid * CAT_ROWS_PW, CAT_ROWS_PW)], sidx)
    cps = [
        pltpu.async_copy(table_hbm.at[sidx.at[j]],
                         rows.at[pl.ds(j * 128, 128)], sem)
        for j in range(CAT_ROWS_PW)
    ]
    for cp in cps:
        cp.wait()
    pltpu.sync_copy(rows, out_hbm.at[pl.ds(wid * CAT_PW, CAT_PW)])


_FB = 1000


def _fuse_body(vt_ref, cat_ref, wv_ref, wc_ref, o_ref):
    dn = (((1,), (1,)), ((), ()))
    o_ref[...] = (
        lax.dot_general(vt_ref[...], wv_ref[...], dn,
                        preferred_element_type=jnp.float32)
        + lax.dot_general(cat_ref[...], wc_ref[...], dn,
                          preferred_element_type=jnp.float32)
    )


def _tc_fuse(vt, cat_emb, wv, wc):
    return pl.pallas_call(
        _fuse_body,
        grid=(N_ITEM // _FB,),
        in_specs=[
            pl.BlockSpec((_FB, VT_DIM), lambda i: (i, 0)),
            pl.BlockSpec((_FB, CAT_EMBED_DIM), lambda i: (i, 0)),
            pl.BlockSpec((HIDDEN, VT_DIM), lambda i: (0, 0)),
            pl.BlockSpec((HIDDEN, CAT_EMBED_DIM), lambda i: (0, 0)),
        ],
        out_specs=pl.BlockSpec((_FB, HIDDEN), lambda i: (i, 0)),
        out_shape=jax.ShapeDtypeStruct((N_ITEM, HIDDEN), jnp.float32),
    )(vt, cat_emb, wv, wc)


def _combine_body(leaky, alo_ref, ahi_ref, deg_ref, x_ref, wll_ref, wlh_ref,
                  wr_ref, b_ref, o_ref):
    dn = (((1,), (1,)), ((), ()))
    inv = 1.0 / jnp.maximum(deg_ref[:, 0:1], 1.0)
    y = (
        lax.dot_general(alo_ref[...] * inv, wll_ref[...], dn,
                        preferred_element_type=jnp.float32)
        + lax.dot_general(ahi_ref[...] * inv, wlh_ref[...], dn,
                          preferred_element_type=jnp.float32)
        + lax.dot_general(x_ref[...], wr_ref[...], dn,
                          preferred_element_type=jnp.float32)
        + b_ref[...]
    )
    if leaky:
        y = jnp.where(y >= 0.0, y, 0.01 * y)
    o_ref[...] = y


def _tc_combine(alo, ahi, deg, x, wl, wr, b, leaky):
    return pl.pallas_call(
        functools.partial(_combine_body, leaky),
        grid=(N_NODES // _FB,),
        in_specs=[
            pl.BlockSpec((_FB, FH), lambda i: (i, 0)),
            pl.BlockSpec((_FB, FH), lambda i: (i, 0)),
            pl.BlockSpec((_FB, LANES), lambda i: (i, 0)),
            pl.BlockSpec((_FB, HIDDEN), lambda i: (i, 0)),
            pl.BlockSpec((HIDDEN, FH), lambda i: (0, 0)),
            pl.BlockSpec((HIDDEN, FH), lambda i: (0, 0)),
            pl.BlockSpec((HIDDEN, HIDDEN), lambda i: (0, 0)),
            pl.BlockSpec((1, HIDDEN), lambda i: (0, 0)),
        ],
        out_specs=pl.BlockSpec((_FB, HIDDEN), lambda i: (i, 0)),
        out_shape=jax.ShapeDtypeStruct((N_NODES, HIDDEN), jnp.float32),
    )(alo, ahi, deg, x, wl[:, :FH], wl[:, FH:], wr, b)


def _unseam(a):
    return jnp.concatenate([a[:HALF], a[ACC:ACC + HALF]], axis=0)


def _halves(a):
    return a[:N_NODES], a[ACC2:ACC2 + N_NODES]


def kernel(vt_feature, cat_indices, cat_offsets, edge_index, cat_table,
           fuse_W, user, conv1_Wl, conv1_Wr, conv1_b, conv2_Wl, conv2_Wr,
           conv2_b):
    del cat_offsets

    src = edge_index[0].astype(jnp.int32)
    dst = edge_index[1].astype(jnp.int32)
    src2d = jnp.pad(src, (0, E_PAD - N_EDGES)).reshape(E_ROWS, 128)
    dst2d = jnp.pad(dst, (0, E_PAD - N_EDGES),
                    constant_values=2 ** 29).reshape(E_ROWS, 128)
    cat2d = jnp.pad(cat_indices.astype(jnp.int32),
                    (0, CAT_PAD - N_ITEM)).reshape(CAT_PAD // 128, 128)

    z32 = jnp.zeros((384, FH), jnp.float32)
    z16 = jnp.zeros((512, LANES), jnp.float32)
    ones16 = jnp.ones((128, LANES), jnp.float32)

    cat_emb = _sc_cat_gather(cat_table, cat2d)
    deg = _unseam(_sc_deg(dst2d, ones16, z16))

    wv = fuse_W[:, :VT_DIM]
    wc = fuse_W[:, VT_DIM:]
    item_feat = _tc_fuse(vt_feature, cat_emb, wv, wc)
    x0 = jnp.concatenate([user, item_feat], axis=0)

    b1 = conv1_b.reshape(1, HIDDEN)
    b2 = conv2_b.reshape(1, HIDDEN)

    x0s = jnp.concatenate([x0[:, :FH], x0[:, FH:]], axis=0)
    a1lo, a1hi = _halves(_sc_spmv(x0s, src2d, dst2d, z32))
    x1 = _tc_combine(a1lo, a1hi, deg, x0, conv1_Wl, conv1_Wr, b1, leaky=True)
    x1s = jnp.concatenate([x1[:, :FH], x1[:, FH:]], axis=0)
    a2lo, a2hi = _halves(_sc_spmv(x1s, src2d, dst2d, z32))
    x2 = _tc_combine(a2lo, a2hi, deg, x1, conv2_Wl, conv2_Wr, b2, leaky=False)
    return x2
